# baseline (speedup 1.0000x reference)
import jax
import jax.numpy as jnp
from jax import lax
from jax.experimental import pallas as pl
from jax.experimental.pallas import tpu as pltpu
from jax.experimental.pallas import tpu_sc as plsc

_L = 16


def _lookup_body(nc: int, nw: int, idx_hbm, out_hbm, buf, table_v):
    b_per_w = idx_hbm.shape[0] // nw
    wid = lax.axis_index("s") * nc + lax.axis_index("c")
    base = wid * b_per_w
    table_v[...] = lax.iota(jnp.int32, _L)
    pltpu.sync_copy(idx_hbm.at[pl.ds(base, b_per_w)], buf)

    def step(i, carry):
        sl = pl.ds(i * _L, _L)
        buf[sl] = plsc.load_gather(table_v, [buf[sl]])
        return carry

    lax.fori_loop(0, b_per_w // _L, step, 0)
    pltpu.sync_copy(buf, out_hbm.at[pl.ds(base, b_per_w)])


def kernel(indices, state):
    b = indices.size
    info = plsc.get_sparse_core_info()
    nw = info.num_cores * info.num_subcores
    b_per_w = b // nw
    flat = indices.reshape(b)
    out = pl.kernel(
        lambda *refs: _lookup_body(info.num_cores, nw, *refs),
        out_type=jax.ShapeDtypeStruct((b,), jnp.int32),
        mesh=plsc.VectorSubcoreMesh(core_axis_name="c", subcore_axis_name="s"),
        compiler_params=pltpu.CompilerParams(needs_layout_passes=False),
        scratch_types=[
            pltpu.VMEM((b_per_w,), jnp.int32),
            pltpu.VMEM((_L,), jnp.int32),
        ],
    )(flat)
    return out.reshape(indices.shape), state


# speedup vs baseline: 1.4726x; 1.4726x over previous
import jax
import jax.numpy as jnp
from jax import lax
from jax.experimental import pallas as pl
from jax.experimental.pallas import tpu as pltpu
from jax.experimental.pallas import tpu_sc as plsc

_L = 16


def _lookup_body(nc: int, nw: int, idx_hbm, out_hbm, buf, table_v):
    b_per_w = idx_hbm.shape[0] // nw
    wid = lax.axis_index("s") * nc + lax.axis_index("c")
    base = wid * b_per_w
    table_v[...] = lax.iota(jnp.int32, _L)
    pltpu.sync_copy(idx_hbm.at[pl.ds(base, b_per_w)], buf)

    @plsc.parallel_loop(0, b_per_w, step=_L, unroll=8)
    def _gstep(i):
        sl = pl.ds(i, _L)
        buf[sl] = plsc.load_gather(table_v, [buf[sl]])
    pltpu.sync_copy(buf, out_hbm.at[pl.ds(base, b_per_w)])


def kernel(indices, state):
    b = indices.size
    info = plsc.get_sparse_core_info()
    nw = info.num_cores * info.num_subcores
    b_per_w = b // nw
    flat = indices.reshape(b)
    out = pl.kernel(
        lambda *refs: _lookup_body(info.num_cores, nw, *refs),
        out_type=jax.ShapeDtypeStruct((b,), jnp.int32),
        mesh=plsc.VectorSubcoreMesh(core_axis_name="c", subcore_axis_name="s"),
        compiler_params=pltpu.CompilerParams(needs_layout_passes=False),
        scratch_types=[
            pltpu.VMEM((b_per_w,), jnp.int32),
            pltpu.VMEM((_L,), jnp.int32),
        ],
    )(flat)
    return out.reshape(indices.shape), state


# speedup vs baseline: 1.5118x; 1.0266x over previous
import jax
import jax.numpy as jnp
from jax import lax
from jax.experimental import pallas as pl
from jax.experimental.pallas import tpu as pltpu
from jax.experimental.pallas import tpu_sc as plsc

_L = 16
_C = 12800


def _lookup_body(nc, nw, nchunks, idx_hbm, out_hbm,
                 ib0, ib1, ob0, ob1, table_v, si0, si1, so0, so1):
    wid = lax.axis_index("s") * nc + lax.axis_index("c")
    base = wid * (nchunks * _C)
    table_v[...] = lax.iota(jnp.int32, _L)
    ibs, obs, sis, sos = (ib0, ib1), (ob0, ob1), (si0, si1), (so0, so1)

    def in_sl(g):
        return idx_hbm.at[pl.ds(base + g * _C, _C)]

    def out_sl(g):
        return out_hbm.at[pl.ds(base + g * _C, _C)]

    pltpu.async_copy(in_sl(0), ibs[0], sis[0])
    pltpu.async_copy(in_sl(1), ibs[1], sis[1])
    for g in range(nchunks):
        b = g % 2
        pltpu.make_async_copy(in_sl(g), ibs[b], sis[b]).wait()
        if g >= 2:
            pltpu.make_async_copy(obs[b], out_sl(g - 2), sos[b]).wait()

        @plsc.parallel_loop(0, _C, step=_L, unroll=8)
        def _gstep(i):
            sl = pl.ds(i, _L)
            obs[b][sl] = plsc.load_gather(table_v, [ibs[b][sl]])

        pltpu.async_copy(obs[b], out_sl(g), sos[b])
        if g + 2 < nchunks:
            pltpu.async_copy(in_sl(g + 2), ibs[b], sis[b])
    for g in range(max(nchunks - 2, 0), nchunks):
        b = g % 2
        pltpu.make_async_copy(obs[b], out_sl(g), sos[b]).wait()


def kernel(indices, state):
    b = indices.size
    info = plsc.get_sparse_core_info()
    nw = info.num_cores * info.num_subcores
    nchunks = b // (nw * _C)
    flat = indices.reshape(b)
    out = pl.kernel(
        lambda *refs: _lookup_body(info.num_cores, nw, nchunks, *refs),
        out_type=jax.ShapeDtypeStruct((b,), jnp.int32),
        mesh=plsc.VectorSubcoreMesh(core_axis_name="c", subcore_axis_name="s"),
        compiler_params=pltpu.CompilerParams(needs_layout_passes=False),
        scratch_types=[
            pltpu.VMEM((_C,), jnp.int32),
            pltpu.VMEM((_C,), jnp.int32),
            pltpu.VMEM((_C,), jnp.int32),
            pltpu.VMEM((_C,), jnp.int32),
            pltpu.VMEM((_L,), jnp.int32),
            pltpu.SemaphoreType.DMA,
            pltpu.SemaphoreType.DMA,
            pltpu.SemaphoreType.DMA,
            pltpu.SemaphoreType.DMA,
        ],
    )(flat)
    return out.reshape(indices.shape), state
